# baseline (speedup 1.0000x reference)
import jax
import jax.numpy as jnp
from jax.experimental import pallas as pl

N = 10000
G = 64


def _head_kernel(pooled_ref, gamma_ref, beta_ref, mean_ref, var_ref, w_ref, b_ref, out_ref):
    pooled = pooled_ref[...]
    bn = (pooled - mean_ref[...]) * jax.lax.rsqrt(var_ref[...] + 1e-5) * gamma_ref[...] + beta_ref[...]
    out_ref[...] = bn @ w_ref[...] + b_ref[...]


def _pna_conv(x, edge_index, W, b):
    src = edge_index[0]
    dst = edge_index[1]
    msg = x[src]
    ones = jnp.ones((msg.shape[0],), jnp.float32)
    deg = jax.ops.segment_sum(ones, dst, num_segments=N)
    s = jax.ops.segment_sum(msg, dst, num_segments=N)
    sq = jax.ops.segment_sum(msg * msg, dst, num_segments=N)
    degc = jnp.maximum(deg, 1.0)[:, None]
    mean = s / degc
    var = jnp.maximum(sq / degc - mean * mean, 0.0)
    std = jnp.sqrt(var + 1e-5)
    mx = jax.ops.segment_max(msg, dst, num_segments=N)
    mn = jax.ops.segment_min(msg, dst, num_segments=N)
    has = (deg > 0)[:, None]
    mx = jnp.where(has, mx, 0.0)
    mn = jnp.where(has, mn, 0.0)
    agg = jnp.concatenate([mean, mn, mx, std], axis=1)
    logd = jnp.log(deg + 1.0)
    delta = jnp.mean(logd)
    amp = (logd / delta)[:, None]
    logd_safe = jnp.where(deg > 0, logd, 1.0)
    att = jnp.where(deg > 0, delta / logd_safe, 1.0)[:, None]
    out = jnp.concatenate([agg, agg * amp, agg * att], axis=1)
    return out @ W + b


def kernel(x, edge_index, batch, W1, b1, W2, b2, W3, b3, bn_gamma, bn_beta, bn_mean, bn_var, fc_W, fc_b):
    h = jax.nn.relu(_pna_conv(x, edge_index, W1, b1))
    h = jax.nn.relu(_pna_conv(h, edge_index, W2, b2))
    h = jax.nn.relu(_pna_conv(h, edge_index, W3, b3))
    pooled = jax.ops.segment_sum(h, batch, num_segments=G)
    out = pl.pallas_call(
        _head_kernel,
        out_shape=jax.ShapeDtypeStruct((G, fc_W.shape[1]), jnp.float32),
    )(pooled, bn_gamma[None, :], bn_beta[None, :], bn_mean[None, :], bn_var[None, :], fc_W, fc_b)
    return out


# speedup vs baseline: 3.7042x; 3.7042x over previous
import functools

import jax
import jax.numpy as jnp
from jax import lax
from jax.experimental import pallas as pl
from jax.experimental.pallas import tpu as pltpu
from jax.experimental.pallas import tpu_sc as plsc

N = 10000
E = 320000
D = 128
G = 64

NR = 64
RS = 160
NPAD = NR * RS

_INFO = plsc.get_sparse_core_info()
NC = _INFO.num_cores
NS = _INFO.num_subcores
NW = NC * NS
RPW = NR // NW

CHUNK = 8000
NCHUNKS = E // CHUNK
BLK = 8192
MAXBLK = E // BLK + 1
CAP = MAXBLK * BLK
CH = 128


def _partition_kernel(src, dst):
    mesh = plsc.VectorSubcoreMesh(core_axis_name="c", subcore_axis_name="s")

    @functools.partial(
        pl.kernel,
        out_type=[jax.ShapeDtypeStruct((NR, CAP), jnp.int32),
                  jax.ShapeDtypeStruct((NR, 16), jnp.int32)],
        mesh=mesh,
        compiler_params=pltpu.CompilerParams(needs_layout_passes=False),
        scratch_types=[
            pltpu.VMEM((CHUNK,), jnp.int32),
            pltpu.VMEM((CHUNK,), jnp.int32),
            pltpu.VMEM((2 * BLK,), jnp.int32),
            pltpu.VMEM((2 * BLK,), jnp.int32),
            pltpu.VMEM((16,), jnp.int32),
        ],
    )
    def part(src_hbm, dst_hbm, lists_hbm, counts_hbm, dbuf, sbuf, stage0,
             stage1, cstage):
        w = lax.axis_index("s") * NC + lax.axis_index("c")
        zero16 = jnp.zeros((16,), jnp.int32)

        def chunk_body(c, carry):
            cnt0, blk0, cnt1, blk1 = carry
            pltpu.sync_copy(dst_hbm.at[pl.ds(c * CHUNK, CHUNK)], dbuf)
            pltpu.sync_copy(src_hbm.at[pl.ds(c * CHUNK, CHUNK)], sbuf)

            def group_body(g, cnts):
                cnt0, cnt1 = cnts
                d = dbuf[pl.ds(g * 16, 16)]
                s = sbuf[pl.ds(g * 16, 16)]
                new = []
                for ri, cntv in ((0, cnt0), (1, cnt1)):
                    rlo = (w * RPW + ri) * RS
                    dloc = d - rlo
                    mask = (dloc >= 0) & (dloc < RS)
                    pos = (plsc.cumsum(mask.astype(jnp.int32)) - 1 + cntv) \
                        & (2 * BLK - 1)
                    rec = s * 512 + dloc
                    plsc.store_scatter((stage0, stage1)[ri], [pos], rec,
                                       mask=mask)
                    new.append(cntv + plsc.all_reduce_population_count(mask))
                return tuple(new)

            cnt0, cnt1 = lax.fori_loop(0, CHUNK // 16, group_body, (cnt0, cnt1))

            outs = []
            for ri, cntv, blk in ((0, cnt0, blk0), (1, cnt1, blk1)):
                r = w * RPW + ri

                stg = (stage0, stage1)[ri]

                def do_drain(blk, stg=stg, r=r):
                    foff = (blk & 1) * BLK
                    pltpu.sync_copy(stg.at[pl.ds(foff, BLK)],
                                    lists_hbm.at[r, pl.ds(blk * BLK, BLK)])
                    return blk + 1

                blk = lax.cond(jnp.max(cntv) - blk * BLK >= BLK, do_drain,
                               lambda b: b, blk)
                outs += [cntv, blk]
            return tuple(outs)

        init = (zero16, jnp.int32(0), zero16, jnp.int32(0))
        cnt0, blk0, cnt1, blk1 = lax.fori_loop(0, NCHUNKS, chunk_body, init)

        for ri, cntv, blk in ((0, cnt0, blk0), (1, cnt1, blk1)):
            r = w * RPW + ri
            foff = (blk & 1) * BLK
            pltpu.sync_copy((stage0, stage1)[ri].at[pl.ds(foff, BLK)],
                            lists_hbm.at[r, pl.ds(blk * BLK, BLK)])
            cstage[...] = cntv
            pltpu.sync_copy(cstage, counts_hbm.at[r])

    return part(src, dst)


def _sc_segment_stats(table, lists, counts):
    mesh = plsc.VectorSubcoreMesh(core_axis_name="c", subcore_axis_name="s")

    @functools.partial(
        pl.kernel,
        out_type=[jax.ShapeDtypeStruct((NPAD, D), jnp.float32)] * 4
        + [jax.ShapeDtypeStruct((NPAD, 16), jnp.float32)],
        mesh=mesh,
        compiler_params=pltpu.CompilerParams(needs_layout_passes=False),
        scratch_types=[
            pltpu.VMEM((RS, D), jnp.float32),
            pltpu.VMEM((RS, D), jnp.float32),
            pltpu.VMEM((RS, D), jnp.float32),
            pltpu.VMEM((RS, D), jnp.float32),
            pltpu.VMEM((RS, 16), jnp.float32),
            pltpu.VMEM((CH,), jnp.int32),
            pltpu.VMEM((1, 128), jnp.int32),
            pltpu.VMEM((CH,), jnp.int32),
            pltpu.VMEM((CH, D), jnp.float32),
            pltpu.VMEM((16,), jnp.int32),
            pltpu.SemaphoreType.DMA,
        ],
    )
    def lk(table_hbm, lists_hbm, counts_hbm, osum, osq, omx, omn, odeg,
           ssum, ssq, smx, smn, sdeg, recbuf, idxbuf, dlocbuf, msgbuf, cbuf,
           sem):
        w = lax.axis_index("s") * NC + lax.axis_index("c")
        zf = jnp.zeros((16,), jnp.float32)
        ninf = jnp.full((16,), -jnp.inf, jnp.float32)
        pinf = jnp.full((16,), jnp.inf, jnp.float32)
        iota = lax.broadcasted_iota(jnp.int32, (16,), 0)

        for ri in range(RPW):
            r = w * RPW + ri

            def zero_body(i, _):
                for j in range(D // 16):
                    ssum[i, pl.ds(j * 16, 16)] = zf
                    ssq[i, pl.ds(j * 16, 16)] = zf
                    smx[i, pl.ds(j * 16, 16)] = ninf
                    smn[i, pl.ds(j * 16, 16)] = pinf
                sdeg[i, pl.ds(0, 16)] = zf
                return 0

            lax.fori_loop(0, RS, zero_body, 0)

            pltpu.sync_copy(counts_hbm.at[r], cbuf)
            cnt = jnp.max(cbuf[...])
            nch = (cnt + CH - 1) // CH

            def chunk_body(c, _):
                base = c * CH
                pltpu.sync_copy(lists_hbm.at[r, pl.ds(base, CH)], recbuf)
                rem = jnp.minimum(cnt - base, CH)
                for k in range(CH // 128):
                    def dec_body(g, _, k=k):
                        off = k * 128 + g * 16
                        rec = recbuf[pl.ds(off, 16)]
                        valid = (base + off + iota) < cnt
                        rec = jnp.where(valid, rec, 0)
                        idxbuf[k, pl.ds(g * 16, 16)] = \
                            lax.shift_right_logical(rec, 9)
                        dlocbuf[pl.ds(off, 16)] = rec & 511
                        return 0

                    lax.fori_loop(0, 8, dec_body, 0)
                    pltpu.async_copy(
                        table_hbm.at[idxbuf.at[k]],
                        msgbuf.at[pl.ds(k * 128, 128), :], sem).wait()

                def edge_body(e, _):
                    dsp = plsc.load_gather(
                        dlocbuf, [jnp.full((16,), e, jnp.int32)])
                    dd = jnp.max(dsp)
                    for j in range(D // 16):
                        m = msgbuf[e, pl.ds(j * 16, 16)]
                        ssum[dd, pl.ds(j * 16, 16)] = \
                            ssum[dd, pl.ds(j * 16, 16)] + m
                        ssq[dd, pl.ds(j * 16, 16)] = \
                            ssq[dd, pl.ds(j * 16, 16)] + m * m
                        smx[dd, pl.ds(j * 16, 16)] = \
                            jnp.maximum(smx[dd, pl.ds(j * 16, 16)], m)
                        smn[dd, pl.ds(j * 16, 16)] = \
                            jnp.minimum(smn[dd, pl.ds(j * 16, 16)], m)
                    sdeg[dd, pl.ds(0, 16)] = sdeg[dd, pl.ds(0, 16)] + 1.0
                    return 0

                lax.fori_loop(0, rem, edge_body, 0)
                return 0

            lax.fori_loop(0, nch, chunk_body, 0)

            pltpu.sync_copy(ssum, osum.at[pl.ds(r * RS, RS), :])
            pltpu.sync_copy(ssq, osq.at[pl.ds(r * RS, RS), :])
            pltpu.sync_copy(smx, omx.at[pl.ds(r * RS, RS), :])
            pltpu.sync_copy(smn, omn.at[pl.ds(r * RS, RS), :])
            pltpu.sync_copy(sdeg, odeg.at[pl.ds(r * RS, RS), :])

    return lk(table, lists, counts)


def _delta_body(deg_ref, out_ref):
    deg = deg_ref[...]
    row = lax.broadcasted_iota(jnp.int32, (NPAD, 16), 0)
    lane = lax.broadcasted_iota(jnp.int32, (NPAD, 16), 1)
    valid = (row < N) & (lane == 0)
    logd = jnp.where(valid, jnp.log(deg + 1.0), 0.0)
    out_ref[...] = jnp.sum(logd).reshape(1, 1) / N


def _delta_kernel(deg):
    return pl.pallas_call(
        _delta_body,
        out_shape=jax.ShapeDtypeStruct((1, 1), jnp.float32),
    )(deg)


_RB = 512


def _finalize_body(s_ref, q_ref, mx_ref, mn_ref, deg_ref, delta_ref,
                   wa_ref, wb_ref, wc_ref, b_ref, out_ref):
    deg = deg_ref[:, 0:1]
    delta = delta_ref[0, 0]
    degc = jnp.maximum(deg, 1.0)
    inv = 1.0 / degc
    mean = s_ref[...] * inv
    var = jnp.maximum(q_ref[...] * inv - mean * mean, 0.0)
    std = jnp.sqrt(var + 1e-5)
    has = deg > 0
    mx = jnp.where(has, mx_ref[...], 0.0)
    mn = jnp.where(has, mn_ref[...], 0.0)
    agg = jnp.concatenate([mean, mn, mx, std], axis=1)
    logd = jnp.log(deg + 1.0)
    amp = logd / delta
    logd_safe = jnp.where(has, logd, 1.0)
    att = jnp.where(has, delta / logd_safe, 1.0)
    acc = jnp.dot(agg, wa_ref[...], preferred_element_type=jnp.float32)
    acc += amp * jnp.dot(agg, wb_ref[...], preferred_element_type=jnp.float32)
    acc += att * jnp.dot(agg, wc_ref[...], preferred_element_type=jnp.float32)
    out_ref[...] = jnp.maximum(acc + b_ref[...], 0.0)


def _finalize_kernel(s, q, mx, mn, deg, delta, W, b):
    wa = W[0:512]
    wb = W[512:1024]
    wc = W[1024:1536]
    grid = NPAD // _RB
    blk = lambda r, c: pl.BlockSpec((_RB, c), lambda i: (i, 0))
    full = lambda rr, cc: pl.BlockSpec((rr, cc), lambda i: (0, 0))
    return pl.pallas_call(
        _finalize_body,
        grid=(grid,),
        in_specs=[blk(_RB, D), blk(_RB, D), blk(_RB, D), blk(_RB, D),
                  blk(_RB, 16), full(1, 1),
                  full(512, D), full(512, D), full(512, D), full(1, D)],
        out_specs=blk(_RB, D),
        out_shape=jax.ShapeDtypeStruct((NPAD, D), jnp.float32),
    )(s, q, mx, mn, deg, delta, wa, wb, wc, b[None, :])


def _head_body(h_ref, batch_ref, gamma_ref, beta_ref, mean_ref, var_ref,
               w_ref, b_ref, out_ref, acc_ref):
    i = pl.program_id(0)

    @pl.when(i == 0)
    def _():
        acc_ref[...] = jnp.zeros_like(acc_ref)

    onehot = (batch_ref[...] == lax.broadcasted_iota(jnp.int32, (_RB, G), 1)
              ).astype(jnp.float32)
    acc_ref[...] += lax.dot_general(onehot, h_ref[...], (((0,), (0,)), ((), ())),
                                    preferred_element_type=jnp.float32)

    @pl.when(i == pl.num_programs(0) - 1)
    def _():
        pooled = acc_ref[...]
        bn = (pooled - mean_ref[...]) * lax.rsqrt(var_ref[...] + 1e-5) \
            * gamma_ref[...] + beta_ref[...]
        out_ref[...] = jnp.dot(bn, w_ref[...], preferred_element_type=jnp.float32) \
            + b_ref[...]


def _head_kernel(h, batch_pad, bn_gamma, bn_beta, bn_mean, bn_var, fc_W, fc_b):
    grid = NPAD // _RB
    blk = lambda c: pl.BlockSpec((_RB, c), lambda i: (i, 0))
    full = lambda rr, cc: pl.BlockSpec((rr, cc), lambda i: (0, 0))
    return pl.pallas_call(
        _head_body,
        grid=(grid,),
        in_specs=[blk(D), blk(1), full(1, D), full(1, D), full(1, D), full(1, D),
                  full(D, G), full(1, G)],
        out_specs=full(G, G),
        out_shape=jax.ShapeDtypeStruct((G, G), jnp.float32),
        scratch_shapes=[pltpu.VMEM((G, D), jnp.float32)],
    )(h, batch_pad, bn_gamma[None, :], bn_beta[None, :], bn_mean[None, :],
      bn_var[None, :], fc_W, fc_b[None, :])


def kernel(x, edge_index, batch, W1, b1, W2, b2, W3, b3, bn_gamma, bn_beta,
           bn_mean, bn_var, fc_W, fc_b):
    batch_pad = jnp.concatenate(
        [batch, jnp.full((NPAD - N,), G, jnp.int32)])[:, None]
    lists, counts = _partition_kernel(edge_index[0], edge_index[1])
    table = x
    delta = None
    deg = None
    for W, b in ((W1, b1), (W2, b2), (W3, b3)):
        s, sq, mx, mn, degrep = _sc_segment_stats(table, lists, counts)
        if delta is None:
            deg = degrep
            delta = _delta_kernel(deg)
        table = _finalize_kernel(s, sq, mx, mn, deg, delta, W, b)
    return _head_kernel(table, batch_pad, bn_gamma, bn_beta, bn_mean, bn_var,
                        fc_W, fc_b)


# speedup vs baseline: 5.8281x; 1.5734x over previous
import functools

import jax
import jax.numpy as jnp
from jax import lax
from jax.experimental import pallas as pl
from jax.experimental.pallas import tpu as pltpu
from jax.experimental.pallas import tpu_sc as plsc

N = 10000
E = 320000
D = 128
G = 64

NR = 96
RS = 112
NPAD = NR * RS

_INFO = plsc.get_sparse_core_info()
NC = _INFO.num_cores
NS = _INFO.num_subcores
NW = NC * NS
RPW = NR // NW

CHUNK = 8000
NCHUNKS = E // CHUNK
BLK = 8192
MAXBLK = E // BLK + 1
CAP = MAXBLK * BLK
CH = 128


def _partition_kernel(src, dst):
    mesh = plsc.VectorSubcoreMesh(core_axis_name="c", subcore_axis_name="s")

    @functools.partial(
        pl.kernel,
        out_type=[jax.ShapeDtypeStruct((NR, CAP), jnp.int32),
                  jax.ShapeDtypeStruct((NR, 16), jnp.int32)],
        mesh=mesh,
        compiler_params=pltpu.CompilerParams(needs_layout_passes=False),
        scratch_types=[
            pltpu.VMEM((CHUNK,), jnp.int32),
            pltpu.VMEM((CHUNK,), jnp.int32),
        ] + [pltpu.VMEM((2 * BLK,), jnp.int32) for _ in range(RPW)]
        + [pltpu.VMEM((16,), jnp.int32)],
    )
    def part(src_hbm, dst_hbm, lists_hbm, counts_hbm, dbuf, sbuf, *stages_c):
        stages = stages_c[:RPW]
        cstage = stages_c[RPW]
        w = lax.axis_index("s") * NC + lax.axis_index("c")
        zero16 = jnp.zeros((16,), jnp.int32)

        def chunk_body(c, carry):
            cnts = carry[:RPW]
            blks = carry[RPW:]
            pltpu.sync_copy(dst_hbm.at[pl.ds(c * CHUNK, CHUNK)], dbuf)
            pltpu.sync_copy(src_hbm.at[pl.ds(c * CHUNK, CHUNK)], sbuf)

            def group_body(g, cnts):
                d = dbuf[pl.ds(g * 16, 16)]
                s = sbuf[pl.ds(g * 16, 16)]
                new = []
                for ri in range(RPW):
                    cntv = cnts[ri]
                    rlo = (w * RPW + ri) * RS
                    dloc = d - rlo
                    mask = (dloc >= 0) & (dloc < RS)
                    pos = (plsc.cumsum(mask.astype(jnp.int32)) - 1 + cntv) \
                        & (2 * BLK - 1)
                    rec = s * 512 + dloc
                    plsc.store_scatter(stages[ri], [pos], rec, mask=mask)
                    new.append(cntv + plsc.all_reduce_population_count(mask))
                return tuple(new)

            cnts = lax.fori_loop(0, CHUNK // 16, group_body, tuple(cnts))

            newblks = []
            for ri in range(RPW):
                r = w * RPW + ri
                stg = stages[ri]

                def do_drain(blk, stg=stg, r=r):
                    foff = (blk & 1) * BLK
                    pltpu.sync_copy(stg.at[pl.ds(foff, BLK)],
                                    lists_hbm.at[r, pl.ds(blk * BLK, BLK)])
                    return blk + 1

                blk = lax.cond(jnp.max(cnts[ri]) - blks[ri] * BLK >= BLK,
                               do_drain, lambda b: b, blks[ri])
                newblks.append(blk)
            return tuple(cnts) + tuple(newblks)

        init = (zero16,) * RPW + (jnp.int32(0),) * RPW
        fin = lax.fori_loop(0, NCHUNKS, chunk_body, init)

        for ri in range(RPW):
            r = w * RPW + ri
            cntv = fin[ri]
            blk = fin[RPW + ri]
            foff = (blk & 1) * BLK
            pltpu.sync_copy(stages[ri].at[pl.ds(foff, BLK)],
                            lists_hbm.at[r, pl.ds(blk * BLK, BLK)])
            cstage[...] = cntv
            pltpu.sync_copy(cstage, counts_hbm.at[r])

    return part(src, dst)


def _sc_segment_stats(table, lists, counts):
    mesh = plsc.VectorSubcoreMesh(core_axis_name="c", subcore_axis_name="s")
    RSD = RS + 1

    @functools.partial(
        pl.kernel,
        out_type=[jax.ShapeDtypeStruct((NPAD, D), jnp.float32)] * 4
        + [jax.ShapeDtypeStruct((NPAD, 16), jnp.float32)],
        mesh=mesh,
        compiler_params=pltpu.CompilerParams(needs_layout_passes=False),
        scratch_types=[
            pltpu.VMEM((RSD, D), jnp.float32),
            pltpu.VMEM((RSD, D), jnp.float32),
            pltpu.VMEM((RSD, D), jnp.float32),
            pltpu.VMEM((RSD, D), jnp.float32),
            pltpu.VMEM((RSD, 16), jnp.float32),
            pltpu.VMEM((2, CH), jnp.int32),
            pltpu.VMEM((CH,), jnp.int32),
            pltpu.VMEM((CH,), jnp.int32),
            pltpu.VMEM((2, CH), jnp.int32),
            pltpu.VMEM((CH, D), jnp.float32),
            pltpu.VMEM((CH, D), jnp.float32),
            pltpu.VMEM((16,), jnp.int32),
            pltpu.SemaphoreType.DMA,
            pltpu.SemaphoreType.DMA,
        ],
    )
    def lk(table_hbm, lists_hbm, counts_hbm, osum, osq, omx, omn, odeg,
           ssum, ssq, smx, smn, sdeg, recbuf, idx0, idx1, dlocbuf, msg0, msg1,
           cbuf, sem0, sem1):
        w = lax.axis_index("s") * NC + lax.axis_index("c")
        zf = jnp.zeros((16,), jnp.float32)
        ninf = jnp.full((16,), -jnp.inf, jnp.float32)
        pinf = jnp.full((16,), jnp.inf, jnp.float32)
        iota = lax.broadcasted_iota(jnp.int32, (16,), 0)
        idxb = (idx0, idx1)
        msgb = (msg0, msg1)
        semb = (sem0, sem1)

        for ri in range(RPW):
            r = w * RPW + ri

            def zero_body(i, _):
                for j in range(D // 16):
                    ssum[i, pl.ds(j * 16, 16)] = zf
                    ssq[i, pl.ds(j * 16, 16)] = zf
                    smx[i, pl.ds(j * 16, 16)] = ninf
                    smn[i, pl.ds(j * 16, 16)] = pinf
                sdeg[i, pl.ds(0, 16)] = zf
                return 0

            lax.fori_loop(0, RSD, zero_body, 0)

            pltpu.sync_copy(counts_hbm.at[r], cbuf)
            cnt = jnp.max(cbuf[...])
            nch = (cnt + CH - 1) // CH

            def stage(c, b):
                pltpu.sync_copy(lists_hbm.at[r, pl.ds(c * CH, CH)],
                                recbuf.at[b])

                def dec_body(g, _):
                    off = g * 16
                    rec = recbuf[b, pl.ds(off, 16)]
                    valid = (c * CH + off + iota) < cnt
                    idxb[b][pl.ds(off, 16)] = jnp.where(
                        valid, lax.shift_right_logical(rec, 9), 0)
                    dlocbuf[b, pl.ds(off, 16)] = jnp.where(
                        valid, rec & 511, RS)
                    return 0

                lax.fori_loop(0, CH // 16, dec_body, 0)
                pltpu.make_async_copy(table_hbm.at[idxb[b]], msgb[b],
                                      semb[b]).start()

            @pl.when(nch > 0)
            def _():
                stage(0, 0)

            def process(c, b):
                pltpu.make_async_copy(table_hbm.at[idxb[b]], msgb[b],
                                      semb[b]).wait()

                def group_body(g, _):
                    dv = dlocbuf[b, pl.ds(g * 16, 16)]
                    for i in range(16):
                        dd = lax.squeeze(lax.slice(dv, (i,), (i + 1,)), (0,))
                        e = g * 16 + i
                        m = [msgb[b][e, pl.ds(j * 16, 16)]
                             for j in range(D // 16)]
                        s0 = [ssum[dd, pl.ds(j * 16, 16)]
                              for j in range(D // 16)]
                        q0 = [ssq[dd, pl.ds(j * 16, 16)]
                              for j in range(D // 16)]
                        x0 = [smx[dd, pl.ds(j * 16, 16)]
                              for j in range(D // 16)]
                        n0 = [smn[dd, pl.ds(j * 16, 16)]
                              for j in range(D // 16)]
                        dg = sdeg[dd, pl.ds(0, 16)]
                        for j in range(D // 16):
                            ssum[dd, pl.ds(j * 16, 16)] = s0[j] + m[j]
                            ssq[dd, pl.ds(j * 16, 16)] = q0[j] + m[j] * m[j]
                            smx[dd, pl.ds(j * 16, 16)] = \
                                jnp.maximum(x0[j], m[j])
                            smn[dd, pl.ds(j * 16, 16)] = \
                                jnp.minimum(n0[j], m[j])
                        sdeg[dd, pl.ds(0, 16)] = dg + 1.0
                    return 0

                lax.fori_loop(0, CH // 16, group_body, 0)

            npairs = (nch + 1) // 2

            def pair_body(p, _):
                for b in range(2):
                    c = 2 * p + b

                    @pl.when(c < nch)
                    def _(c=c, b=b):
                        @pl.when(c + 1 < nch)
                        def _():
                            stage(c + 1, 1 - b)

                        process(c, b)
                return 0

            lax.fori_loop(0, npairs, pair_body, 0)

            row0 = pl.multiple_of(r * RS, 8)
            pltpu.sync_copy(ssum.at[pl.ds(0, RS), :],
                            osum.at[pl.ds(row0, RS), :])
            pltpu.sync_copy(ssq.at[pl.ds(0, RS), :],
                            osq.at[pl.ds(row0, RS), :])
            pltpu.sync_copy(smx.at[pl.ds(0, RS), :],
                            omx.at[pl.ds(row0, RS), :])
            pltpu.sync_copy(smn.at[pl.ds(0, RS), :],
                            omn.at[pl.ds(row0, RS), :])
            pltpu.sync_copy(sdeg.at[pl.ds(0, RS), :],
                            odeg.at[pl.ds(row0, RS), :])

    return lk(table, lists, counts)


def _delta_body(deg_ref, out_ref):
    deg = deg_ref[...]
    row = lax.broadcasted_iota(jnp.int32, (NPAD, 16), 0)
    lane = lax.broadcasted_iota(jnp.int32, (NPAD, 16), 1)
    valid = (row < N) & (lane == 0)
    logd = jnp.where(valid, jnp.log(deg + 1.0), 0.0)
    out_ref[...] = jnp.sum(logd).reshape(1, 1) / N


def _delta_kernel(deg):
    return pl.pallas_call(
        _delta_body,
        out_shape=jax.ShapeDtypeStruct((1, 1), jnp.float32),
    )(deg)


_RB = 512


def _finalize_body(s_ref, q_ref, mx_ref, mn_ref, deg_ref, delta_ref,
                   wa_ref, wb_ref, wc_ref, b_ref, out_ref):
    deg = deg_ref[:, 0:1]
    delta = delta_ref[0, 0]
    degc = jnp.maximum(deg, 1.0)
    inv = 1.0 / degc
    mean = s_ref[...] * inv
    var = jnp.maximum(q_ref[...] * inv - mean * mean, 0.0)
    std = jnp.sqrt(var + 1e-5)
    has = deg > 0
    mx = jnp.where(has, mx_ref[...], 0.0)
    mn = jnp.where(has, mn_ref[...], 0.0)
    agg = jnp.concatenate([mean, mn, mx, std], axis=1)
    logd = jnp.log(deg + 1.0)
    amp = logd / delta
    logd_safe = jnp.where(has, logd, 1.0)
    att = jnp.where(has, delta / logd_safe, 1.0)
    acc = jnp.dot(agg, wa_ref[...], preferred_element_type=jnp.float32)
    acc += amp * jnp.dot(agg, wb_ref[...], preferred_element_type=jnp.float32)
    acc += att * jnp.dot(agg, wc_ref[...], preferred_element_type=jnp.float32)
    out_ref[...] = jnp.maximum(acc + b_ref[...], 0.0)


def _finalize_kernel(s, q, mx, mn, deg, delta, W, b):
    wa = W[0:512]
    wb = W[512:1024]
    wc = W[1024:1536]
    grid = NPAD // _RB
    blk = lambda r, c: pl.BlockSpec((_RB, c), lambda i: (i, 0))
    full = lambda rr, cc: pl.BlockSpec((rr, cc), lambda i: (0, 0))
    return pl.pallas_call(
        _finalize_body,
        grid=(grid,),
        in_specs=[blk(_RB, D), blk(_RB, D), blk(_RB, D), blk(_RB, D),
                  blk(_RB, 16), full(1, 1),
                  full(512, D), full(512, D), full(512, D), full(1, D)],
        out_specs=blk(_RB, D),
        out_shape=jax.ShapeDtypeStruct((NPAD, D), jnp.float32),
    )(s, q, mx, mn, deg, delta, wa, wb, wc, b[None, :])


def _head_body(h_ref, batch_ref, gamma_ref, beta_ref, mean_ref, var_ref,
               w_ref, b_ref, out_ref, acc_ref):
    i = pl.program_id(0)

    @pl.when(i == 0)
    def _():
        acc_ref[...] = jnp.zeros_like(acc_ref)

    onehot = (batch_ref[...] == lax.broadcasted_iota(jnp.int32, (_RB, G), 1)
              ).astype(jnp.float32)
    acc_ref[...] += lax.dot_general(onehot, h_ref[...], (((0,), (0,)), ((), ())),
                                    preferred_element_type=jnp.float32)

    @pl.when(i == pl.num_programs(0) - 1)
    def _():
        pooled = acc_ref[...]
        bn = (pooled - mean_ref[...]) * lax.rsqrt(var_ref[...] + 1e-5) \
            * gamma_ref[...] + beta_ref[...]
        out_ref[...] = jnp.dot(bn, w_ref[...], preferred_element_type=jnp.float32) \
            + b_ref[...]


def _head_kernel(h, batch_pad, bn_gamma, bn_beta, bn_mean, bn_var, fc_W, fc_b):
    grid = NPAD // _RB
    blk = lambda c: pl.BlockSpec((_RB, c), lambda i: (i, 0))
    full = lambda rr, cc: pl.BlockSpec((rr, cc), lambda i: (0, 0))
    return pl.pallas_call(
        _head_body,
        grid=(grid,),
        in_specs=[blk(D), blk(1), full(1, D), full(1, D), full(1, D), full(1, D),
                  full(D, G), full(1, G)],
        out_specs=full(G, G),
        out_shape=jax.ShapeDtypeStruct((G, G), jnp.float32),
        scratch_shapes=[pltpu.VMEM((G, D), jnp.float32)],
    )(h, batch_pad, bn_gamma[None, :], bn_beta[None, :], bn_mean[None, :],
      bn_var[None, :], fc_W, fc_b[None, :])


def kernel(x, edge_index, batch, W1, b1, W2, b2, W3, b3, bn_gamma, bn_beta,
           bn_mean, bn_var, fc_W, fc_b):
    batch_pad = jnp.concatenate(
        [batch, jnp.full((NPAD - N,), G, jnp.int32)])[:, None]
    lists, counts = _partition_kernel(edge_index[0], edge_index[1])
    table = x
    delta = None
    deg = None
    for W, b in ((W1, b1), (W2, b2), (W3, b3)):
        s, sq, mx, mn, degrep = _sc_segment_stats(table, lists, counts)
        if delta is None:
            deg = degrep
            delta = _delta_kernel(deg)
        table = _finalize_kernel(s, sq, mx, mn, deg, delta, W, b)
    return _head_kernel(table, batch_pad, bn_gamma, bn_beta, bn_mean, bn_var,
                        fc_W, fc_b)
